# Initial kernel scaffold; baseline (speedup 1.0000x reference)
import jax
import jax.numpy as jnp
from jax.experimental import pallas as pl


def kernel(z_e, codebook):
    raise NotImplementedError("write your pallas kernel here")


# speedup vs baseline: 1.1707x; 1.1707x over previous
import functools

import jax
import jax.numpy as jnp
from jax import lax
from jax.experimental import pallas as pl
from jax.experimental.pallas import tpu as pltpu
from jax.experimental.pallas import tpu_sc as plsc

NUM_EMB = 8192
DIM = 64
M_TILE = 1024
N_TILE = 1024


def _argmin_body(zT_ref, cb_ref, zn_ref, cn_ref, out_ref, val_ref):
    j = pl.program_id(1)

    @pl.when(j == 0)
    def _():
        val_ref[...] = jnp.full_like(val_ref, jnp.inf)

    zT = zT_ref[...]
    cb = cb_ref[...]
    s = lax.dot_general(cb, zT, dimension_numbers=(((1,), (0,)), ((), ())),
                        preferred_element_type=jnp.float32)
    zn = zn_ref[0]
    cn = cn_ref[...]
    d = (zn - 2.0 * s) + cn
    v = jnp.min(d, axis=0, keepdims=True)
    rows = lax.broadcasted_iota(jnp.int32, d.shape, 0)
    ii = jnp.min(jnp.where(d == v, rows, NUM_EMB), axis=0, keepdims=True)
    ii = ii + j * N_TILE
    bv = val_ref[...]
    upd = v < bv
    val_ref[...] = jnp.where(upd, v, bv)
    @pl.when(j == 0)
    def _():
        out_ref[0] = ii
    @pl.when(j > 0)
    def _():
        out_ref[0] = jnp.where(upd, ii, out_ref[0])


def _argmin_indices(zT, cb, zn3, cn2):
    m_blocks = zT.shape[1] // M_TILE
    n_blocks = NUM_EMB // N_TILE
    grid = (m_blocks, n_blocks)
    return pl.pallas_call(
        _argmin_body,
        grid=grid,
        in_specs=[
            pl.BlockSpec((DIM, M_TILE), lambda m, j: (0, m)),
            pl.BlockSpec((N_TILE, DIM), lambda m, j: (j, 0)),
            pl.BlockSpec((1, 1, M_TILE), lambda m, j: (m, 0, 0)),
            pl.BlockSpec((N_TILE, 1), lambda m, j: (j, 0)),
        ],
        out_specs=pl.BlockSpec((1, 1, M_TILE), lambda m, j: (m, 0, 0)),
        out_shape=jax.ShapeDtypeStruct((m_blocks, 1, M_TILE), jnp.int32),
        scratch_shapes=[pltpu.VMEM((1, M_TILE), jnp.float32)],
    )(zT, cb, zn3, cn2)


def _make_gather(batch, width):
    info = plsc.get_sparse_core_info()
    nw = info.num_cores * info.num_subcores
    b_per_w = batch // nw
    mesh = plsc.VectorSubcoreMesh(core_axis_name="c", subcore_axis_name="s")

    @functools.partial(
        pl.kernel,
        out_type=jax.ShapeDtypeStruct((batch, width), jnp.float32),
        mesh=mesh,
        scratch_types=[
            pltpu.VMEM((b_per_w,), jnp.int32),
            pltpu.VMEM((b_per_w, width), jnp.float32),
            pltpu.SemaphoreType.DMA,
        ],
    )
    def gather(table_hbm, idx_hbm, out_hbm, idx_v, rows_v, sem):
        wid = lax.axis_index("s") * info.num_cores + lax.axis_index("c")
        base = wid * b_per_w
        pltpu.sync_copy(idx_hbm.at[pl.ds(base, b_per_w)], idx_v)
        pltpu.async_copy(table_hbm.at[idx_v], rows_v, sem).wait()
        pltpu.sync_copy(rows_v, out_hbm.at[pl.ds(base, b_per_w)])

    return gather


def kernel(z_e, codebook):
    b, c, h, w = z_e.shape
    batch = b * h * w
    z = jnp.transpose(z_e, (0, 2, 3, 1))
    z_flat = z.reshape(-1, DIM)
    zn = jnp.sum(z_flat ** 2, axis=1, keepdims=True)
    cn = jnp.sum(codebook ** 2, axis=1)
    zT = z_flat.T
    zn3 = zn.reshape(batch // M_TILE, 1, M_TILE)
    cn2 = cn.reshape(NUM_EMB, 1)

    idx3 = _argmin_indices(zT, codebook, zn3, cn2)
    indices = idx3.reshape(batch)

    cb_pad = jnp.pad(codebook, ((0, 0), (0, 128 - DIM)))
    zq_pad = _make_gather(batch, 128)(cb_pad, indices)
    z_q = jnp.transpose(zq_pad.reshape(b, h, w, 128)[..., :DIM], (0, 3, 1, 2))
    return (z_q, z_q, indices)


# speedup vs baseline: 1.2463x; 1.0645x over previous
import functools

import jax
import jax.numpy as jnp
from jax import lax
from jax.experimental import pallas as pl
from jax.experimental.pallas import tpu as pltpu
from jax.experimental.pallas import tpu_sc as plsc

NUM_EMB = 8192
DIM = 64
M_TILE = 1024
N_TILE = 1024


def _argmin_body(z3_ref, cb2_ref, zn_ref, cn_ref, rows_ref, out_ref, val_ref):
    j = pl.program_id(1)

    @pl.when(j == 0)
    def _():
        val_ref[...] = jnp.full_like(val_ref, jnp.inf)

    zT = z3_ref[0]
    cb2 = cb2_ref[...]
    s2 = lax.dot_general(cb2, zT, dimension_numbers=(((1,), (0,)), ((), ())),
                         preferred_element_type=jnp.float32)
    zn = zn_ref[0]
    cn = cn_ref[...]
    d = (zn - s2) + cn
    v = jnp.min(d, axis=0, keepdims=True)
    rows = rows_ref[...]
    ii_f = jnp.min(jnp.where(d == v, rows, float(NUM_EMB)),
                   axis=0, keepdims=True)
    ii = ii_f.astype(jnp.int32)
    bv = val_ref[...]
    upd = v < bv
    val_ref[...] = jnp.where(upd, v, bv)
    @pl.when(j == 0)
    def _():
        out_ref[0] = ii
    @pl.when(j > 0)
    def _():
        out_ref[0] = jnp.where(upd, ii, out_ref[0])


def _argmin_indices(z3, cb2, zn3, cn2, rowsf):
    m_blocks = z3.shape[0]
    n_blocks = NUM_EMB // N_TILE
    grid = (m_blocks, n_blocks)
    return pl.pallas_call(
        _argmin_body,
        grid=grid,
        in_specs=[
            pl.BlockSpec((1, DIM, M_TILE), lambda m, j: (m, 0, 0)),
            pl.BlockSpec((N_TILE, DIM), lambda m, j: (j, 0)),
            pl.BlockSpec((1, 1, M_TILE), lambda m, j: (m, 0, 0)),
            pl.BlockSpec((N_TILE, 1), lambda m, j: (j, 0)),
            pl.BlockSpec((N_TILE, 1), lambda m, j: (j, 0)),
        ],
        out_specs=pl.BlockSpec((1, 1, M_TILE), lambda m, j: (m, 0, 0)),
        out_shape=jax.ShapeDtypeStruct((m_blocks, 1, M_TILE), jnp.int32),
        scratch_shapes=[pltpu.VMEM((1, M_TILE), jnp.float32)],
    )(z3, cb2, zn3, cn2, rowsf)


def _make_gather(batch, width):
    info = plsc.get_sparse_core_info()
    nw = info.num_cores * info.num_subcores
    b_per_w = batch // nw
    mesh = plsc.VectorSubcoreMesh(core_axis_name="c", subcore_axis_name="s")

    @functools.partial(
        pl.kernel,
        out_type=jax.ShapeDtypeStruct((batch, width), jnp.float32),
        mesh=mesh,
        scratch_types=[
            pltpu.VMEM((b_per_w,), jnp.int32),
            pltpu.VMEM((b_per_w, width), jnp.float32),
            pltpu.SemaphoreType.DMA,
        ],
    )
    def gather(table_hbm, idx_hbm, out_hbm, idx_v, rows_v, sem):
        wid = lax.axis_index("s") * info.num_cores + lax.axis_index("c")
        base = wid * b_per_w
        pltpu.sync_copy(idx_hbm.at[pl.ds(base, b_per_w)], idx_v)
        pltpu.async_copy(table_hbm.at[idx_v], rows_v, sem).wait()
        pltpu.sync_copy(rows_v, out_hbm.at[pl.ds(base, b_per_w)])

    return gather


def kernel(z_e, codebook):
    b, c, h, w = z_e.shape
    batch = b * h * w
    z = jnp.transpose(z_e, (0, 2, 3, 1))
    z_flat = z.reshape(-1, DIM)
    zn = jnp.sum(z_flat ** 2, axis=1, keepdims=True)
    cn = jnp.sum(codebook ** 2, axis=1)
    z3 = z_e.reshape(b, c, h * w)
    zn3 = zn.reshape(batch // M_TILE, 1, M_TILE)
    cn2 = cn.reshape(NUM_EMB, 1)
    cb2 = codebook + codebook
    rowsf = jnp.arange(NUM_EMB, dtype=jnp.float32).reshape(NUM_EMB, 1)

    idx3 = _argmin_indices(z3, cb2, zn3, cn2, rowsf)
    indices = idx3.reshape(batch)

    cb_pad = jnp.pad(codebook, ((0, 0), (0, 128 - DIM)))
    zq_pad = _make_gather(batch, 128)(cb_pad, indices)
    z_q = jnp.transpose(zq_pad.reshape(b, h, w, 128)[..., :DIM], (0, 3, 1, 2))
    return (z_q, z_q, indices)


# speedup vs baseline: 1.2576x; 1.0091x over previous
import functools

import jax
import jax.numpy as jnp
from jax import lax
from jax.experimental import pallas as pl
from jax.experimental.pallas import tpu as pltpu
from jax.experimental.pallas import tpu_sc as plsc

NUM_EMB = 8192
DIM = 64
M_TILE = 1024
N_TILE = 1024


def _argmin_body(z3_ref, cb2_ref, zn_ref, cn_ref, rows_ref, out_ref, val_ref):
    j = pl.program_id(1)

    @pl.when(j == 0)
    def _():
        val_ref[...] = jnp.full_like(val_ref, jnp.inf)

    zT = z3_ref[0].astype(jnp.bfloat16)
    cb2 = cb2_ref[...].astype(jnp.bfloat16)
    s2 = lax.dot_general(cb2, zT, dimension_numbers=(((1,), (0,)), ((), ())),
                         preferred_element_type=jnp.float32)
    zn = zn_ref[0]
    cn = cn_ref[...]
    d = (zn - s2) + cn
    v = jnp.min(d, axis=0, keepdims=True)
    rows = rows_ref[...]
    ii_f = jnp.min(jnp.where(d == v, rows, float(NUM_EMB)),
                   axis=0, keepdims=True)
    ii = ii_f.astype(jnp.int32)
    bv = val_ref[...]
    upd = v < bv
    val_ref[...] = jnp.where(upd, v, bv)
    @pl.when(j == 0)
    def _():
        out_ref[0] = ii
    @pl.when(j > 0)
    def _():
        out_ref[0] = jnp.where(upd, ii, out_ref[0])


def _argmin_indices(z3, cb2, zn3, cn2, rowsf):
    m_blocks = z3.shape[0]
    n_blocks = NUM_EMB // N_TILE
    grid = (m_blocks, n_blocks)
    return pl.pallas_call(
        _argmin_body,
        grid=grid,
        in_specs=[
            pl.BlockSpec((1, DIM, M_TILE), lambda m, j: (m, 0, 0)),
            pl.BlockSpec((N_TILE, DIM), lambda m, j: (j, 0)),
            pl.BlockSpec((1, 1, M_TILE), lambda m, j: (m, 0, 0)),
            pl.BlockSpec((N_TILE, 1), lambda m, j: (j, 0)),
            pl.BlockSpec((N_TILE, 1), lambda m, j: (j, 0)),
        ],
        out_specs=pl.BlockSpec((1, 1, M_TILE), lambda m, j: (m, 0, 0)),
        out_shape=jax.ShapeDtypeStruct((m_blocks, 1, M_TILE), jnp.int32),
        scratch_shapes=[pltpu.VMEM((1, M_TILE), jnp.float32)],
    )(z3, cb2, zn3, cn2, rowsf)


def _make_gather(batch, width):
    info = plsc.get_sparse_core_info()
    nw = info.num_cores * info.num_subcores
    b_per_w = batch // nw
    mesh = plsc.VectorSubcoreMesh(core_axis_name="c", subcore_axis_name="s")

    @functools.partial(
        pl.kernel,
        out_type=jax.ShapeDtypeStruct((batch, width), jnp.float32),
        mesh=mesh,
        scratch_types=[
            pltpu.VMEM((b_per_w,), jnp.int32),
            pltpu.VMEM((b_per_w, width), jnp.float32),
            pltpu.SemaphoreType.DMA,
        ],
    )
    def gather(table_hbm, idx_hbm, out_hbm, idx_v, rows_v, sem):
        wid = lax.axis_index("s") * info.num_cores + lax.axis_index("c")
        base = wid * b_per_w
        pltpu.sync_copy(idx_hbm.at[pl.ds(base, b_per_w)], idx_v)
        pltpu.async_copy(table_hbm.at[idx_v], rows_v, sem).wait()
        pltpu.sync_copy(rows_v, out_hbm.at[pl.ds(base, b_per_w)])

    return gather


def kernel(z_e, codebook):
    b, c, h, w = z_e.shape
    batch = b * h * w
    z = jnp.transpose(z_e, (0, 2, 3, 1))
    z_flat = z.reshape(-1, DIM)
    zn = jnp.sum(z_flat ** 2, axis=1, keepdims=True)
    cn = jnp.sum(codebook ** 2, axis=1)
    z3 = z_e.reshape(b, c, h * w)
    zn3 = zn.reshape(batch // M_TILE, 1, M_TILE)
    cn2 = cn.reshape(NUM_EMB, 1)
    cb2 = codebook + codebook
    rowsf = jnp.arange(NUM_EMB, dtype=jnp.float32).reshape(NUM_EMB, 1)

    idx3 = _argmin_indices(z3, cb2, zn3, cn2, rowsf)
    indices = idx3.reshape(batch)

    cb_pad = jnp.pad(codebook, ((0, 0), (0, 128 - DIM)))
    zq_pad = _make_gather(batch, 128)(cb_pad, indices)
    z_q = jnp.transpose(zq_pad.reshape(b, h, w, 128)[..., :DIM], (0, 3, 1, 2))
    return (z_q, z_q, indices)


# speedup vs baseline: 1.3475x; 1.0715x over previous
import functools

import jax
import jax.numpy as jnp
from jax import lax
from jax.experimental import pallas as pl
from jax.experimental.pallas import tpu as pltpu
from jax.experimental.pallas import tpu_sc as plsc

NUM_EMB = 8192
DIM = 64
M_TILE = 1024
N_TILE = 1024


def _argmin_body(z3_ref, cb_ref, out_ref, val_ref, zn_ref, cn_ref, rows_ref):
    m = pl.program_id(0)
    j = pl.program_id(1)

    zT = z3_ref[0]
    cb = cb_ref[...]
    cb2 = cb + cb

    @pl.when(j == 0)
    def _():
        val_ref[...] = jnp.full_like(val_ref, jnp.inf)
        zn_ref[...] = jnp.sum(zT * zT, axis=0, keepdims=True)

    @pl.when(m == 0)
    def _():
        cn_ref[pl.ds(pl.multiple_of(j * N_TILE, N_TILE), N_TILE), :] = (
            0.25 * jnp.sum(cb2 * cb2, axis=1, keepdims=True))

    @pl.when(jnp.logical_and(m == 0, j == 0))
    def _():
        riota = lax.broadcasted_iota(jnp.int32, (N_TILE, 1), 0)
        rows_ref[...] = riota.astype(jnp.float32)

    s2 = lax.dot_general(cb2.astype(jnp.bfloat16), zT.astype(jnp.bfloat16),
                         dimension_numbers=(((1,), (0,)), ((), ())),
                         preferred_element_type=jnp.float32)
    zn = zn_ref[...]
    cn = cn_ref[pl.ds(pl.multiple_of(j * N_TILE, N_TILE), N_TILE), :]
    d = (zn - s2) + cn
    v = jnp.min(d, axis=0, keepdims=True)
    rows = rows_ref[...]
    ii_f = jnp.min(jnp.where(d == v, rows, float(N_TILE)),
                   axis=0, keepdims=True)
    ii = ii_f.astype(jnp.int32) + j * N_TILE
    bv = val_ref[...]
    upd = v < bv
    val_ref[...] = jnp.where(upd, v, bv)
    @pl.when(j == 0)
    def _():
        out_ref[0] = ii
    @pl.when(j > 0)
    def _():
        out_ref[0] = jnp.where(upd, ii, out_ref[0])


def _argmin_indices(z3, codebook):
    m_blocks = z3.shape[0]
    n_blocks = NUM_EMB // N_TILE
    grid = (m_blocks, n_blocks)
    return pl.pallas_call(
        _argmin_body,
        grid=grid,
        in_specs=[
            pl.BlockSpec((1, DIM, M_TILE), lambda m, j: (m, 0, 0)),
            pl.BlockSpec((N_TILE, DIM), lambda m, j: (j, 0)),
        ],
        out_specs=pl.BlockSpec((1, 1, M_TILE), lambda m, j: (m, 0, 0)),
        out_shape=jax.ShapeDtypeStruct((m_blocks, 1, M_TILE), jnp.int32),
        scratch_shapes=[
            pltpu.VMEM((1, M_TILE), jnp.float32),
            pltpu.VMEM((1, M_TILE), jnp.float32),
            pltpu.VMEM((NUM_EMB, 1), jnp.float32),
            pltpu.VMEM((N_TILE, 1), jnp.float32),
        ],
    )(z3, codebook)


def _make_gather(batch, width):
    info = plsc.get_sparse_core_info()
    nw = info.num_cores * info.num_subcores
    b_per_w = batch // nw
    mesh = plsc.VectorSubcoreMesh(core_axis_name="c", subcore_axis_name="s")

    @functools.partial(
        pl.kernel,
        out_type=jax.ShapeDtypeStruct((batch, width), jnp.float32),
        mesh=mesh,
        scratch_types=[
            pltpu.VMEM((b_per_w,), jnp.int32),
            pltpu.VMEM((b_per_w, width), jnp.float32),
            pltpu.SemaphoreType.DMA,
        ],
    )
    def gather(table_hbm, idx_hbm, out_hbm, idx_v, rows_v, sem):
        wid = lax.axis_index("s") * info.num_cores + lax.axis_index("c")
        base = wid * b_per_w
        pltpu.sync_copy(idx_hbm.at[pl.ds(base, b_per_w)], idx_v)
        pltpu.async_copy(table_hbm.at[idx_v], rows_v, sem).wait()
        pltpu.sync_copy(rows_v, out_hbm.at[pl.ds(base, b_per_w)])

    return gather


def kernel(z_e, codebook):
    b, c, h, w = z_e.shape
    batch = b * h * w
    z3 = z_e.reshape(b, c, h * w)

    idx3 = _argmin_indices(z3, codebook)
    indices = idx3.reshape(batch)

    cb_pad = jnp.pad(codebook, ((0, 0), (0, 128 - DIM)))
    zq_pad = _make_gather(batch, 128)(cb_pad, indices)
    z_q = jnp.transpose(zq_pad.reshape(b, h, w, 128)[..., :DIM], (0, 3, 1, 2))
    return (z_q, z_q, indices)


# speedup vs baseline: 1.4229x; 1.0559x over previous
import functools

import jax
import jax.numpy as jnp
from jax import lax
from jax.experimental import pallas as pl
from jax.experimental.pallas import tpu as pltpu
from jax.experimental.pallas import tpu_sc as plsc

NUM_EMB = 8192
DIM = 64
M_TILE = 1024
N_TILE = 2048


def _argmin_body(z3_ref, cb_ref, out_ref, val_ref, zn_ref, cn_ref, rows_ref):
    m = pl.program_id(0)
    j = pl.program_id(1)

    zT = z3_ref[0]
    cb = cb_ref[...]
    cb2 = cb + cb

    @pl.when(j == 0)
    def _():
        val_ref[...] = jnp.full_like(val_ref, jnp.inf)
        zn_ref[...] = jnp.sum(zT * zT, axis=0, keepdims=True)

    @pl.when(m == 0)
    def _():
        cn_ref[pl.ds(pl.multiple_of(j * N_TILE, N_TILE), N_TILE), :] = (
            0.25 * jnp.sum(cb2 * cb2, axis=1, keepdims=True))

    @pl.when(jnp.logical_and(m == 0, j == 0))
    def _():
        riota = lax.broadcasted_iota(jnp.int32, (N_TILE, 1), 0)
        rows_ref[...] = riota.astype(jnp.float32)

    s2 = lax.dot_general(cb2.astype(jnp.bfloat16), zT.astype(jnp.bfloat16),
                         dimension_numbers=(((1,), (0,)), ((), ())),
                         preferred_element_type=jnp.float32)
    zn = zn_ref[...]
    cn = cn_ref[pl.ds(pl.multiple_of(j * N_TILE, N_TILE), N_TILE), :]
    d = (zn - s2) + cn
    v = jnp.min(d, axis=0, keepdims=True)
    rows = rows_ref[...]
    ii_f = jnp.min(jnp.where(d == v, rows, float(N_TILE)),
                   axis=0, keepdims=True)
    ii = ii_f.astype(jnp.int32) + j * N_TILE
    bv = val_ref[...]
    upd = v < bv
    val_ref[...] = jnp.where(upd, v, bv)
    @pl.when(j == 0)
    def _():
        out_ref[0] = ii
    @pl.when(j > 0)
    def _():
        out_ref[0] = jnp.where(upd, ii, out_ref[0])


def _argmin_indices(z3, codebook):
    m_blocks = z3.shape[0]
    n_blocks = NUM_EMB // N_TILE
    grid = (m_blocks, n_blocks)
    return pl.pallas_call(
        _argmin_body,
        grid=grid,
        in_specs=[
            pl.BlockSpec((1, DIM, M_TILE), lambda m, j: (m, 0, 0)),
            pl.BlockSpec((N_TILE, DIM), lambda m, j: (j, 0)),
        ],
        out_specs=pl.BlockSpec((1, 1, M_TILE), lambda m, j: (m, 0, 0)),
        out_shape=jax.ShapeDtypeStruct((m_blocks, 1, M_TILE), jnp.int32),
        scratch_shapes=[
            pltpu.VMEM((1, M_TILE), jnp.float32),
            pltpu.VMEM((1, M_TILE), jnp.float32),
            pltpu.VMEM((NUM_EMB, 1), jnp.float32),
            pltpu.VMEM((N_TILE, 1), jnp.float32),
        ],
    )(z3, codebook)


def _make_gather(batch, width):
    info = plsc.get_sparse_core_info()
    nw = info.num_cores * info.num_subcores
    b_per_w = batch // nw
    mesh = plsc.VectorSubcoreMesh(core_axis_name="c", subcore_axis_name="s")

    @functools.partial(
        pl.kernel,
        out_type=jax.ShapeDtypeStruct((batch, width), jnp.float32),
        mesh=mesh,
        scratch_types=[
            pltpu.VMEM((b_per_w,), jnp.int32),
            pltpu.VMEM((b_per_w, width), jnp.float32),
            pltpu.SemaphoreType.DMA,
        ],
    )
    def gather(table_hbm, idx_hbm, out_hbm, idx_v, rows_v, sem):
        wid = lax.axis_index("s") * info.num_cores + lax.axis_index("c")
        base = wid * b_per_w
        pltpu.sync_copy(idx_hbm.at[pl.ds(base, b_per_w)], idx_v)
        pltpu.async_copy(table_hbm.at[idx_v], rows_v, sem).wait()
        pltpu.sync_copy(rows_v, out_hbm.at[pl.ds(base, b_per_w)])

    return gather


def kernel(z_e, codebook):
    b, c, h, w = z_e.shape
    batch = b * h * w
    z3 = z_e.reshape(b, c, h * w)

    idx3 = _argmin_indices(z3, codebook)
    indices = idx3.reshape(batch)

    cb_pad = jnp.pad(codebook, ((0, 0), (0, 128 - DIM)))
    zq_pad = _make_gather(batch, 128)(cb_pad, indices)
    z_q = jnp.transpose(zq_pad.reshape(b, h, w, 128)[..., :DIM], (0, 3, 1, 2))
    return (z_q, z_q, indices)


# speedup vs baseline: 1.4839x; 1.0429x over previous
import functools

import jax
import jax.numpy as jnp
from jax import lax
from jax.experimental import pallas as pl
from jax.experimental.pallas import tpu as pltpu
from jax.experimental.pallas import tpu_sc as plsc

NUM_EMB = 8192
DIM = 64
M_TILE = 1024
N_TILE = 4096


def _argmin_body(z3_ref, cb_ref, out_ref, val_ref, zn_ref, cn_ref, rows_ref):
    m = pl.program_id(0)
    j = pl.program_id(1)

    zT = z3_ref[0]
    cb = cb_ref[...]
    cb2 = cb + cb

    @pl.when(j == 0)
    def _():
        val_ref[...] = jnp.full_like(val_ref, jnp.inf)
        zn_ref[...] = jnp.sum(zT * zT, axis=0, keepdims=True)

    @pl.when(m == 0)
    def _():
        cn_ref[pl.ds(pl.multiple_of(j * N_TILE, N_TILE), N_TILE), :] = (
            0.25 * jnp.sum(cb2 * cb2, axis=1, keepdims=True))

    @pl.when(jnp.logical_and(m == 0, j == 0))
    def _():
        riota = lax.broadcasted_iota(jnp.int32, (N_TILE, 1), 0)
        rows_ref[...] = riota.astype(jnp.float32)

    s2 = lax.dot_general(cb2.astype(jnp.bfloat16), zT.astype(jnp.bfloat16),
                         dimension_numbers=(((1,), (0,)), ((), ())),
                         preferred_element_type=jnp.float32)
    zn = zn_ref[...]
    cn = cn_ref[pl.ds(pl.multiple_of(j * N_TILE, N_TILE), N_TILE), :]
    d = (zn - s2) + cn
    v = jnp.min(d, axis=0, keepdims=True)
    rows = rows_ref[...]
    ii_f = jnp.min(jnp.where(d == v, rows, float(N_TILE)),
                   axis=0, keepdims=True)
    ii = ii_f.astype(jnp.int32) + j * N_TILE
    bv = val_ref[...]
    upd = v < bv
    val_ref[...] = jnp.where(upd, v, bv)
    @pl.when(j == 0)
    def _():
        out_ref[0] = ii
    @pl.when(j > 0)
    def _():
        out_ref[0] = jnp.where(upd, ii, out_ref[0])


def _argmin_indices(z3, codebook):
    m_blocks = z3.shape[0]
    n_blocks = NUM_EMB // N_TILE
    grid = (m_blocks, n_blocks)
    return pl.pallas_call(
        _argmin_body,
        grid=grid,
        in_specs=[
            pl.BlockSpec((1, DIM, M_TILE), lambda m, j: (m, 0, 0)),
            pl.BlockSpec((N_TILE, DIM), lambda m, j: (j, 0)),
        ],
        out_specs=pl.BlockSpec((1, 1, M_TILE), lambda m, j: (m, 0, 0)),
        out_shape=jax.ShapeDtypeStruct((m_blocks, 1, M_TILE), jnp.int32),
        scratch_shapes=[
            pltpu.VMEM((1, M_TILE), jnp.float32),
            pltpu.VMEM((1, M_TILE), jnp.float32),
            pltpu.VMEM((NUM_EMB, 1), jnp.float32),
            pltpu.VMEM((N_TILE, 1), jnp.float32),
        ],
    )(z3, codebook)


def _make_gather(batch, width):
    info = plsc.get_sparse_core_info()
    nw = info.num_cores * info.num_subcores
    b_per_w = batch // nw
    mesh = plsc.VectorSubcoreMesh(core_axis_name="c", subcore_axis_name="s")

    @functools.partial(
        pl.kernel,
        out_type=jax.ShapeDtypeStruct((batch, width), jnp.float32),
        mesh=mesh,
        scratch_types=[
            pltpu.VMEM((b_per_w,), jnp.int32),
            pltpu.VMEM((b_per_w, width), jnp.float32),
            pltpu.SemaphoreType.DMA,
        ],
    )
    def gather(table_hbm, idx_hbm, out_hbm, idx_v, rows_v, sem):
        wid = lax.axis_index("s") * info.num_cores + lax.axis_index("c")
        base = wid * b_per_w
        pltpu.sync_copy(idx_hbm.at[pl.ds(base, b_per_w)], idx_v)
        pltpu.async_copy(table_hbm.at[idx_v], rows_v, sem).wait()
        pltpu.sync_copy(rows_v, out_hbm.at[pl.ds(base, b_per_w)])

    return gather


def kernel(z_e, codebook):
    b, c, h, w = z_e.shape
    batch = b * h * w
    z3 = z_e.reshape(b, c, h * w)

    idx3 = _argmin_indices(z3, codebook)
    indices = idx3.reshape(batch)

    cb_pad = jnp.pad(codebook, ((0, 0), (0, 128 - DIM)))
    zq_pad = _make_gather(batch, 128)(cb_pad, indices)
    z_q = jnp.transpose(zq_pad.reshape(b, h, w, 128)[..., :DIM], (0, 3, 1, 2))
    return (z_q, z_q, indices)
